# baseline (speedup 1.0000x reference)
import functools

import jax
import jax.numpy as jnp
from jax import lax
from jax.experimental import pallas as pl
from jax.experimental.pallas import tpu as pltpu
from jax.experimental.pallas import tpu_sc as plsc

_NC = 2
_NS = 16
_NW = _NC * _NS
_CC = 4


def _tree_sum(vs):
    while len(vs) > 1:
        nxt = [vs[i] + vs[i + 1] for i in range(0, len(vs) - 1, 2)]
        if len(vs) % 2:
            nxt.append(vs[-1])
        vs = nxt
    return vs[0]


@functools.partial(jax.jit, static_argnames=("b", "c", "t", "h", "w"))
def _avg_pool(x, b, c, t, h, w):
    n = b * c * t
    pairs = b * c
    ppw = pairs // _NW
    spw = ppw * t
    nch = ppw // _CC
    wpb = c // ppw
    n_valid = (h - 1) * (h - 2)
    inv = 1.0 / float(n_valid)

    mesh = plsc.VectorSubcoreMesh(core_axis_name="c", subcore_axis_name="s")

    @functools.partial(
        pl.kernel,
        out_type=jax.ShapeDtypeStruct((n,), jnp.float32),
        mesh=mesh,
        compiler_params=pltpu.CompilerParams(use_tc_tiling_on_sc=False),
        scratch_types=[
            pltpu.VMEM((_CC, t, h, w), jnp.float32),
            pltpu.VMEM((_CC, t, h, w), jnp.float32),
            pltpu.VMEM((spw,), jnp.float32),
            pltpu.SemaphoreType.DMA,
            pltpu.SemaphoreType.DMA,
        ],
    )
    def sc_kernel(x_hbm, out_hbm, buf0, buf1, outbuf, sem0, sem1):
        wid = lax.axis_index("s") * _NC + lax.axis_index("c")
        bi = wid // wpb
        c0 = (wid % wpb) * ppw

        lane = lax.iota(jnp.int32, 16)
        wvec = jnp.where((lane >= 1) & (lane <= 14), inv, 0.0).astype(
            jnp.float32
        )
        perms = {k: lane ^ k for k in (1, 2, 4, 8)}
        sels = {k: (lane & k) != 0 for k in (1, 2, 4, 8)}

        def merge(a, bb, k):
            pa = a + a.at[perms[k]].get(mode="promise_in_bounds")
            pb = bb + bb.at[perms[k]].get(mode="promise_in_bounds")
            return jnp.where(sels[k], pb, pa)

        def start(ci, buf, sem):
            pltpu.async_copy(
                x_hbm.at[bi, pl.ds(c0 + ci * _CC, _CC)], buf, sem
            )

        def wait(ci, buf, sem):
            pltpu.make_async_copy(
                x_hbm.at[bi, pl.ds(c0 + ci * _CC, _CC)], buf, sem
            ).wait()

        def compute(buf, ci):
            @pl.loop(0, _CC * t // 16)
            def _grp(gi):
                cc = gi // (t // 16)
                t0 = (gi % (t // 16)) * 16
                accs = []
                for jj in range(16):
                    rows = [buf[cc, t0 + jj, r] for r in range(h - 1)]
                    accs.append(_tree_sum(rows) * wvec)
                vs = accs
                for k in (1, 2, 4, 8):
                    vs = [
                        merge(vs[2 * i], vs[2 * i + 1], k)
                        for i in range(len(vs) // 2)
                    ]
                outbuf[pl.ds(ci * _CC * t + gi * 16, 16)] = vs[0]

        start(0, buf0, sem0)

        @pl.loop(0, nch, step=2)
        def _chunk(ci):
            start(ci + 1, buf1, sem1)
            wait(ci, buf0, sem0)
            compute(buf0, ci)

            @pl.when(ci + 2 < nch)
            def _():
                start(ci + 2, buf0, sem0)

            wait(ci + 1, buf1, sem1)
            compute(buf1, ci + 1)

        pltpu.sync_copy(outbuf, out_hbm.at[pl.ds(wid * spw, spw)])

    return sc_kernel(x).reshape(b, c, t)


def kernel(x):
    b, c, t, h, w = x.shape
    assert h == 16 and w == 16, "kernel specialized to 16x16 spatial grids"
    pairs = b * c
    assert pairs % _NW == 0 and (pairs // _NW) % (2 * _CC) == 0
    assert t % 16 == 0
    return _avg_pool(x, b, c, t, h, w)


# speedup vs baseline: 11.2098x; 11.2098x over previous
import functools

import jax
import jax.numpy as jnp
from jax import lax
from jax.experimental import pallas as pl
from jax.experimental.pallas import tpu as pltpu
from jax.experimental.pallas import tpu_sc as plsc

_NC = 2
_NS = 16
_NW = _NC * _NS


@functools.partial(jax.jit, static_argnames=("b", "c", "t", "h", "w"))
def _avg_pool(x, b, c, t, h, w):
    xt = lax.transpose(x, (0, 2, 3, 4, 1))
    n = b * c * t
    pairs = b * t
    ppw = pairs // _NW
    nch = 2 * ppw
    cg = c // 16
    n_valid = (h - 1) * (h - 2)
    inv = 1.0 / float(n_valid)

    mesh = plsc.VectorSubcoreMesh(core_axis_name="c", subcore_axis_name="s")

    @functools.partial(
        pl.kernel,
        out_type=jax.ShapeDtypeStruct((n,), jnp.float32),
        mesh=mesh,
        scratch_types=[
            pltpu.VMEM((8, h, c), jnp.float32),
            pltpu.VMEM((8, h, c), jnp.float32),
            pltpu.VMEM((ppw * c,), jnp.float32),
            pltpu.SemaphoreType.DMA,
            pltpu.SemaphoreType.DMA,
        ],
    )
    def sc_kernel(xt_hbm, out_hbm, buf0, buf1, outbuf, sem0, sem1):
        wid = lax.axis_index("s") * _NC + lax.axis_index("c")
        p0 = wid * ppw

        def src(ci):
            p = p0 + ci // 2
            h0 = (ci % 2) * 7
            return xt_hbm.at[p // t, p % t, pl.ds(h0, 8)]

        def start(ci, buf, sem):
            pltpu.async_copy(src(ci), buf, sem)

        def wait(ci, buf, sem):
            pltpu.make_async_copy(src(ci), buf, sem).wait()

        def compute(buf, ci, first):
            pi = ci // 2
            hs = range(8) if first else range(1, 8)

            @pl.loop(0, cg)
            def _cgrp(k):
                o = pl.ds(pi * c + k * 16, 16)
                vs = [
                    buf[hh, ww, pl.ds(k * 16, 16)]
                    for hh in hs
                    for ww in range(1, w - 1)
                ]
                if not first:
                    vs.append(outbuf[o])
                while len(vs) > 1:
                    nxt = [
                        vs[i] + vs[i + 1] for i in range(0, len(vs) - 1, 2)
                    ]
                    if len(vs) % 2:
                        nxt.append(vs[-1])
                    vs = nxt
                outbuf[o] = vs[0] if first else vs[0] * inv

        start(0, buf0, sem0)

        @pl.loop(0, nch, step=2)
        def _chunk(ci):
            start(ci + 1, buf1, sem1)
            wait(ci, buf0, sem0)
            compute(buf0, ci, True)

            @pl.when(ci + 2 < nch)
            def _():
                start(ci + 2, buf0, sem0)

            wait(ci + 1, buf1, sem1)
            compute(buf1, ci + 1, False)

        pltpu.sync_copy(outbuf, out_hbm.at[pl.ds(p0 * c, ppw * c)])

    out = sc_kernel(xt)
    return lax.transpose(out.reshape(b, t, c), (0, 2, 1))


def kernel(x):
    b, c, t, h, w = x.shape
    assert h == 16 and w == 16, "kernel specialized to 16x16 spatial grids"
    assert (b * t) % _NW == 0 and c % 16 == 0
    return _avg_pool(x, b, c, t, h, w)
